# baseline (speedup 1.0000x reference)
import functools

import jax
import jax.numpy as jnp
from jax import lax
from jax.experimental import pallas as pl
from jax.experimental.pallas import tpu as pltpu
from jax.experimental.pallas import tpu_sc as plsc

B, L, D = 4096, 200, 64
N = B * L
NP = N // 2

_info = plsc.get_sparse_core_info()
NC, NS = _info.num_cores, _info.num_subcores
NW = NC * NS
PPW = NP // NW
CP = 800
NCHUNK = PPW // CP

_mesh = plsc.VectorSubcoreMesh(core_axis_name="c", subcore_axis_name="s")


@functools.partial(
    pl.kernel,
    mesh=_mesh,
    out_type=jax.ShapeDtypeStruct((NP, 2 * D), jnp.float32),
    scratch_types=[
        pltpu.VMEM((CP,), jnp.int32),
        pltpu.VMEM((CP, 2 * D), jnp.float32),
        pltpu.VMEM_SHARED((16, 2 * D), jnp.float32),
        pltpu.SemaphoreType.DMA,
    ],
)
def _sc_lookup(pidx_hbm, table2_hbm, out_hbm, pidx_v, rows_v, table_s, sem):
    wid = lax.axis_index("s") * NC + lax.axis_index("c")
    pltpu.sync_copy(table2_hbm, table_s)

    def body(i, carry):
        base = wid * PPW + i * CP
        pltpu.sync_copy(pidx_hbm.at[pl.ds(base, CP)], pidx_v)
        pltpu.async_copy(table_s.at[pidx_v], rows_v, sem).wait()
        pltpu.sync_copy(rows_v, out_hbm.at[pl.ds(base, CP)])
        return carry

    lax.fori_loop(0, NCHUNK, body, 0)


def kernel(mask, emb_weight):
    flat = mask.reshape(NP, 2).astype(jnp.int32)
    pair_codes = flat[:, 0] * 4 + flat[:, 1]
    left = jnp.repeat(emb_weight, 4, axis=0)
    right = jnp.tile(emb_weight, (4, 1))
    table2 = jnp.concatenate([left, right], axis=1)
    out = _sc_lookup(pair_codes, table2)
    return out.reshape(B, L, D)


# speedup vs baseline: 1.0806x; 1.0806x over previous
import functools

import jax
import jax.numpy as jnp
from jax import lax
from jax.experimental import pallas as pl
from jax.experimental.pallas import tpu as pltpu
from jax.experimental.pallas import tpu_sc as plsc

B, L, D = 4096, 200, 64
N = B * L
NP = N // 2

_info = plsc.get_sparse_core_info()
NC, NS = _info.num_cores, _info.num_subcores
NW = NC * NS
PPW = NP // NW
CP = 800
NCHUNK = PPW // CP

_mesh = plsc.VectorSubcoreMesh(core_axis_name="c", subcore_axis_name="s")


@functools.partial(
    pl.kernel,
    mesh=_mesh,
    out_type=jax.ShapeDtypeStruct((NP, 2 * D), jnp.float32),
    scratch_types=[
        pltpu.VMEM((CP,), jnp.int32),
        pltpu.VMEM((CP, 2 * D), jnp.float32),
        pltpu.VMEM_SHARED((16, 2 * D), jnp.float32),
        pltpu.SemaphoreType.DMA,
    ],
)
def _sc_lookup(pidx_hbm, table2_hbm, out_hbm, pidx_v, rows_v, table_s, sem):
    wid = lax.axis_index("s") * NC + lax.axis_index("c")
    pltpu.sync_copy(table2_hbm, table_s)

    def body(i, carry):
        base = wid * PPW + i * CP
        pltpu.sync_copy(pidx_hbm.at[pl.ds(base, CP)], pidx_v)
        pltpu.async_copy(table_s.at[pidx_v], rows_v, sem).wait()
        pltpu.sync_copy(rows_v, out_hbm.at[pl.ds(base, CP)])
        return carry

    lax.fori_loop(0, NCHUNK, body, 0)


def kernel(mask, emb_weight):
    flat = mask.reshape(N).astype(jnp.int32)
    pair_codes = flat[0::2] * 4 + flat[1::2]
    left = jnp.repeat(emb_weight, 4, axis=0)
    right = jnp.tile(emb_weight, (4, 1))
    table2 = jnp.concatenate([left, right], axis=1)
    out = _sc_lookup(pair_codes, table2)
    return out.reshape(B, L, D)


# speedup vs baseline: 1.7947x; 1.6609x over previous
import functools

import jax
import jax.numpy as jnp
from jax import lax
from jax.experimental import pallas as pl
from jax.experimental.pallas import tpu as pltpu
from jax.experimental.pallas import tpu_sc as plsc

B, L, D = 4096, 200, 64
N = B * L

_info = plsc.get_sparse_core_info()
NC, NS = _info.num_cores, _info.num_subcores
NW = NC * NS
BPW = B // NW
CB = 4
NCHUNK = BPW // CB
CIDX = CB * L

_mesh = plsc.VectorSubcoreMesh(core_axis_name="c", subcore_axis_name="s")


@functools.partial(
    pl.kernel,
    mesh=_mesh,
    out_type=jax.ShapeDtypeStruct((N, 2 * D), jnp.float32),
    scratch_types=[
        pltpu.VMEM((CIDX,), jnp.int32),
        pltpu.VMEM((CIDX, 2 * D), jnp.float32),
        pltpu.VMEM_SHARED((4, 2 * D), jnp.float32),
        pltpu.SemaphoreType.DMA,
    ],
)
def _sc_lookup(mask_hbm, table_hbm, out_hbm, idx_v, rows_v, table_v, sem):
    wid = lax.axis_index("s") * NC + lax.axis_index("c")
    pltpu.sync_copy(table_hbm, table_v)

    def body(i, carry):
        b0 = wid * BPW + i * CB
        pltpu.sync_copy(mask_hbm.at[pl.ds(b0 * L, CIDX)], idx_v)
        pltpu.async_copy(table_v.at[idx_v], rows_v, sem).wait()
        pltpu.sync_copy(rows_v, out_hbm.at[pl.ds(b0 * L, CIDX)])
        return carry

    lax.fori_loop(0, NCHUNK, body, 0)


def kernel(mask, emb_weight):
    flat = mask.reshape(N).astype(jnp.int32)
    wpad = jnp.concatenate(
        [emb_weight, jnp.zeros((4, D), jnp.float32)], axis=1)
    out = _sc_lookup(flat, wpad)
    return out[:, :D].reshape(B, L, D)
